# baseline (speedup 1.0000x reference)
import functools

import jax
import jax.numpy as jnp
from jax import lax
from jax.experimental import pallas as pl
from jax.experimental.pallas import tpu as pltpu
from jax.experimental.pallas import tpu_sc as plsc

NC = 2
NS = 16
NW = NC * NS
D = 128
CHUNK = 128
NBUF = 5


def _flat_gather(x_flat, weight):
    total = x_flat.shape[0]
    per_w = total // NW
    nchunk = per_w // CHUNK

    mesh = plsc.VectorSubcoreMesh(core_axis_name="c", subcore_axis_name="s")

    scratch = [
        pltpu.VMEM((per_w,), jnp.int32),
        pltpu.VMEM((NBUF, CHUNK, D), jnp.float32),
    ] + [pltpu.SemaphoreType.DMA] * (2 * NBUF)

    @functools.partial(
        pl.kernel,
        mesh=mesh,
        out_type=jax.ShapeDtypeStruct((total, D), jnp.float32),
        scratch_types=scratch,
    )
    def emb(x_hbm, w_hbm, out_hbm, idx_v, rows_v, *sems):
        gsems, osems = sems[:NBUF], sems[NBUF:]
        wid = lax.axis_index("s") * NC + lax.axis_index("c")
        base = pl.multiple_of(wid * per_w, CHUNK)
        pltpu.sync_copy(x_hbm.at[pl.ds(base, per_w)], idx_v)

        def idx_slice(g):
            return idx_v.at[pl.ds(pl.multiple_of(g * CHUNK, CHUNK), CHUNK)]

        def out_slice(g):
            return out_hbm.at[pl.ds(pl.multiple_of(base + g * CHUNK, CHUNK), CHUNK)]

        def gather(g, b):
            return pltpu.make_async_copy(w_hbm.at[idx_slice(g)], rows_v.at[b], gsems[b])

        def write(g, b):
            return pltpu.make_async_copy(rows_v.at[b], out_slice(g), osems[b])

        def outer(go, carry):
            for b in range(NBUF):
                g = go * NBUF + b

                @pl.when(g >= NBUF)
                def _():
                    write(g - NBUF, b).wait()

                gather(g, b).start()

                gc = g - (NBUF - 1)
                bc = (b + 1) % NBUF

                @pl.when(gc >= 0)
                def _():
                    gather(gc, bc).wait()
                    write(gc, bc).start()

            return carry

        lax.fori_loop(0, nchunk // NBUF, outer, 0)

        for k in range(NBUF - 1):
            gc = nchunk - (NBUF - 1) + k
            gather(gc, gc % NBUF).wait()
            write(gc, gc % NBUF).start()
        for b in range(NBUF):
            write(nchunk - NBUF + b, b).wait()

    return emb(x_flat, weight)


def kernel(x, weight):
    b, s = x.shape
    out = _flat_gather(x.reshape(b * s), weight)
    return out.reshape(b, s, weight.shape[1])


# speedup vs baseline: 1.7894x; 1.7894x over previous
import functools

import jax
import jax.numpy as jnp
from jax import lax
from jax.experimental import pallas as pl
from jax.experimental.pallas import tpu as pltpu
from jax.experimental.pallas import tpu_sc as plsc

NC = 2
NS = 16
NW = NC * NS
NBUF = 8


def _make_emb(n_rows, seq, vocab, d):
    rows_per_w = n_rows // NW

    mesh = plsc.VectorSubcoreMesh(core_axis_name="c", subcore_axis_name="s")

    scratch = [
        pltpu.VMEM((rows_per_w, seq), jnp.int32),
        pltpu.VMEM((NBUF, seq, d), jnp.float32),
    ] + [pltpu.SemaphoreType.DMA] * (2 * NBUF)

    @functools.partial(
        pl.kernel,
        mesh=mesh,
        out_type=jax.ShapeDtypeStruct((n_rows, seq, d), jnp.float32),
        scratch_types=scratch,
    )
    def emb(x_hbm, w_hbm, out_hbm, idx_v, rows_v, *sems):
        gsems, osems = sems[:NBUF], sems[NBUF:]
        wid = lax.axis_index("s") * NC + lax.axis_index("c")
        base = wid * rows_per_w
        pltpu.sync_copy(x_hbm.at[pl.ds(base, rows_per_w)], idx_v)

        def gather(g, b):
            return pltpu.make_async_copy(
                w_hbm.at[idx_v.at[g]], rows_v.at[b], gsems[b]
            )

        def write(g, b):
            return pltpu.make_async_copy(rows_v.at[b], out_hbm.at[base + g], osems[b])

        def outer(go, carry):
            for b in range(NBUF):
                g = go * NBUF + b

                @pl.when(g >= NBUF)
                def _():
                    write(g - NBUF, b).wait()

                gather(g, b).start()

                gc = g - (NBUF - 1)
                bc = (b + 1) % NBUF

                @pl.when(gc >= 0)
                def _():
                    gather(gc, bc).wait()
                    write(gc, bc).start()

            return carry

        lax.fori_loop(0, rows_per_w // NBUF, outer, 0)

        for k in range(NBUF - 1):
            gc = rows_per_w - (NBUF - 1) + k
            gather(gc, gc % NBUF).wait()
            write(gc, gc % NBUF).start()
        for b in range(NBUF):
            write(rows_per_w - NBUF + b, b).wait()

    return emb


def kernel(x, weight):
    n_rows, seq = x.shape
    vocab, d = weight.shape
    return _make_emb(n_rows, seq, vocab, d)(x, weight)
